# baseline (speedup 1.0000x reference)
import jax
import jax.numpy as jnp
from jax import lax
from jax.experimental import pallas as pl
from jax.experimental.pallas import tpu as pltpu
from jax.experimental.pallas import tpu_sc as plsc

N, C, D, H, W = 2, 3, 128, 128, 128
LOCS = N * D * H * W
VOL = D * H * W
NW = 32
PER_W = LOCS // NW
CHUNK = 4096
NCHUNK = PER_W // CHUNK
ROWS = CHUNK // 128

_MAGIC = 12582912.0


def _body(xt8, fx_h, fy_h, fz_h, sx_h, sy_h, sz_h, out_h,
          in_v, idx_v, vals_v, sem_in, sem_g, sem_o):
    c_id = lax.axis_index("c")
    s_id = lax.axis_index("s")
    wbase = c_id * VOL + s_id * PER_W
    hbm_ins = (fx_h, fy_h, fz_h, sx_h, sy_h, sz_h)

    def fire_in(k, b):
        base = wbase + k * CHUNK
        for i, h in enumerate(hbm_ins):
            pltpu.async_copy(h.at[pl.ds(base, CHUNK)], in_v.at[b, i], sem_in)

    def wait_in(b):
        for i, h in enumerate(hbm_ins):
            pltpu.make_async_copy(h.at[pl.ds(0, CHUNK)], in_v.at[b, i],
                                  sem_in).wait()

    def compute(b):
        @pl.loop(0, ROWS)
        def _rows(j):
            for l in range(8):
                s_ = pl.ds(j * 128 + l * 16, 16)

                def to_idx(g, size):
                    t = ((g + 1.0) * 0.5) * float(size - 1)
                    r = (t + _MAGIC) - _MAGIC
                    r = jnp.minimum(jnp.maximum(r, 0.0), float(size - 1))
                    return r.astype(jnp.int32)

                ix = to_idx(in_v[b, 0, s_] + in_v[b, 3, s_], W)
                iy = to_idx(in_v[b, 1, s_] + in_v[b, 4, s_], H)
                iz = to_idx(in_v[b, 2, s_] + in_v[b, 5, s_], D)
                idx_v[b, j, pl.ds(l * 16, 16)] = (
                    iz * (H * W) + iy * W + ix + c_id * VOL)

    def fire_gather(b):
        @pl.loop(0, ROWS)
        def _fire(j):
            pltpu.async_copy(xt8.at[idx_v.at[b, j]],
                             vals_v.at[b, pl.ds(j * 128, 128)], sem_g)

    def drain_gather(b):
        @pl.loop(0, ROWS)
        def _drain(j):
            pltpu.make_async_copy(xt8.at[idx_v.at[b, j]],
                                  vals_v.at[b, pl.ds(j * 128, 128)],
                                  sem_g).wait()

    def fire_out(k, b):
        base = pl.multiple_of(wbase + k * CHUNK, 8)
        pltpu.async_copy(vals_v.at[b], out_h.at[pl.ds(base, CHUNK)], sem_o)

    def wait_out(b):
        pltpu.make_async_copy(xt8.at[pl.ds(0, CHUNK)], vals_v.at[b],
                              sem_o).wait()

    fire_in(0, 0)
    wait_in(0)
    compute(0)

    @pl.loop(0, NCHUNK // 2)
    def _super(m):
        for half in range(2):
            k = m * 2 + half
            b = half
            nb = 1 - half
            fire_gather(b)
            fire_in((k + 1) % NCHUNK, nb)
            wait_in(nb)
            compute(nb)

            @pl.when(k > 0)
            def _():
                wait_out(nb)
            drain_gather(b)
            fire_out(k, b)

    wait_out(1)


_grid_sample_sc = pl.kernel(
    _body,
    out_type=jax.ShapeDtypeStruct((LOCS, 8), jnp.float32),
    mesh=plsc.VectorSubcoreMesh(core_axis_name="c", subcore_axis_name="s"),
    compiler_params=pltpu.CompilerParams(use_tc_tiling_on_sc=False),
    scratch_types=[
        pltpu.VMEM((2, 6, CHUNK), jnp.float32),
        pltpu.VMEM((2, ROWS, 128), jnp.int32),
        pltpu.VMEM((2, CHUNK, 8), jnp.float32),
        pltpu.SemaphoreType.DMA,
        pltpu.SemaphoreType.DMA,
        pltpu.SemaphoreType.DMA,
    ],
)


def kernel(x, flow, sample_grid):
    assert x.shape == (N, C, D, H, W)
    xt = x.transpose(0, 2, 3, 4, 1).reshape(LOCS, C)
    xt8 = jnp.pad(xt, ((0, 0), (0, 8 - C)))
    fx = flow[..., 0].reshape(-1)
    fy = flow[..., 1].reshape(-1)
    fz = flow[..., 2].reshape(-1)
    sx = sample_grid[..., 0].reshape(-1)
    sy = sample_grid[..., 1].reshape(-1)
    sz = sample_grid[..., 2].reshape(-1)
    out8 = _grid_sample_sc(xt8, fx, fy, fz, sx, sy, sz)
    out = out8.reshape(N, D, H, W, 8)[..., :C]
    return out.transpose(0, 4, 1, 2, 3)


# speedup vs baseline: 12.1958x; 12.1958x over previous
import jax
import jax.numpy as jnp
from jax import lax
from jax.experimental import pallas as pl
from jax.experimental.pallas import tpu as pltpu
from jax.experimental.pallas import tpu_sc as plsc

N, C, D, H, W = 2, 3, 128, 128, 128
LOCS = N * D * H * W
VOL = D * H * W
NW = 32
PER_W = LOCS // NW
CHUNK = 2048
NCHUNK = PER_W // CHUNK
ROWS = CHUNK // 128
OROWS = CHUNK // 128

_MAGIC = 12582912.0


def _body(x2, fx_h, fy_h, fz_h, sx_h, sy_h, sz_h, out_h, xt8_h,
          in_v, idx_v, vals_v, och_v, sem_in, sem_g, sem_o):
    c_id = lax.axis_index("c")
    s_id = lax.axis_index("s")
    wbase = c_id * VOL + s_id * PER_W
    hbm_ins = (fx_h, fy_h, fz_h, sx_h, sy_h, sz_h)
    iota16 = lax.iota(jnp.int32, 16)

    def p0_fire_in(k, b):
        base = s_id * PER_W + k * CHUNK
        for ch in range(C):
            pltpu.async_copy(x2.at[c_id * C + ch, pl.ds(base, CHUNK)],
                             in_v.at[b, ch], sem_in)

    def p0_wait_in(b):
        for ch in range(C):
            pltpu.make_async_copy(x2.at[0, pl.ds(0, CHUNK)], in_v.at[b, ch],
                                  sem_in).wait()

    def p0_interleave(b):
        @pl.loop(0, CHUNK // 16)
        def _grp(l):
            ridx = l * 16 + iota16
            for ch in range(C):
                v = in_v[b, ch, pl.ds(l * 16, 16)]
                cidx = jnp.full((16,), ch, dtype=jnp.int32)
                plsc.store_scatter(vals_v.at[b], [ridx, cidx], v)

    def p0_fire_out(k, b):
        base = pl.multiple_of(wbase + k * CHUNK, 8)
        pltpu.async_copy(vals_v.at[b], xt8_h.at[pl.ds(base, CHUNK)], sem_o)

    def p0_wait_out_simple(b):
        pltpu.make_async_copy(xt8_h.at[pl.ds(0, CHUNK)], vals_v.at[b],
                              sem_o).wait()

    p0_fire_in(0, 0)

    @pl.loop(0, NCHUNK // 2)
    def _p0(m):
        for half in range(2):
            k = m * 2 + half
            b = half
            nb = 1 - half
            p0_wait_in(b)
            p0_fire_in((k + 1) % NCHUNK, nb)

            @pl.when(k >= 2)
            def _():
                p0_wait_out_simple(b)
            p0_interleave(b)
            p0_fire_out(k, b)

    p0_wait_in(0)
    p0_wait_out_simple(0)
    p0_wait_out_simple(1)
    plsc.subcore_barrier()

    def fire_in(k, b):
        base = wbase + k * CHUNK
        for i, h in enumerate(hbm_ins):
            pltpu.async_copy(h.at[pl.ds(base, CHUNK)], in_v.at[b, i], sem_in)

    def wait_in(b):
        for i, h in enumerate(hbm_ins):
            pltpu.make_async_copy(h.at[pl.ds(0, CHUNK)], in_v.at[b, i],
                                  sem_in).wait()

    def compute(b):
        @pl.loop(0, ROWS)
        def _rows(j):
            for l in range(8):
                s_ = pl.ds(j * 128 + l * 16, 16)

                def to_idx(g, size):
                    t = ((g + 1.0) * 0.5) * float(size - 1)
                    r = (t + _MAGIC) - _MAGIC
                    r = jnp.minimum(jnp.maximum(r, 0.0), float(size - 1))
                    return r.astype(jnp.int32)

                ix = to_idx(in_v[b, 0, s_] + in_v[b, 3, s_], W)
                iy = to_idx(in_v[b, 1, s_] + in_v[b, 4, s_], H)
                iz = to_idx(in_v[b, 2, s_] + in_v[b, 5, s_], D)
                idx_v[b, j, pl.ds(l * 16, 16)] = (
                    iz * (H * W) + iy * W + ix + c_id * VOL)

    def fire_gather(b):
        @pl.loop(0, ROWS)
        def _fire(j):
            pltpu.async_copy(xt8_h.at[idx_v.at[b, j]],
                             vals_v.at[b, pl.ds(j * 128, 128)], sem_g)

    def drain_gather(b):
        @pl.loop(0, ROWS)
        def _drain(j):
            pltpu.make_async_copy(xt8_h.at[idx_v.at[b, j]],
                                  vals_v.at[b, pl.ds(j * 128, 128)],
                                  sem_g).wait()

    def deinterleave(b):
        @pl.loop(0, OROWS)
        def _drow(j):
            for l in range(8):
                ridx = (j * 128 + l * 16) + iota16
                for ch in range(C):
                    cidx = jnp.full((16,), ch, dtype=jnp.int32)
                    v = plsc.load_gather(vals_v.at[b], [ridx, cidx])
                    och_v[b, ch, j, pl.ds(l * 16, 16)] = v

    def out_row0(k, ch):
        return pl.multiple_of(
            (c_id * (C * VOL) + ch * VOL + s_id * PER_W + k * CHUNK) // 128, 8)

    def fire_out(k, b):
        for ch in range(C):
            pltpu.async_copy(och_v.at[b, ch],
                             out_h.at[pl.ds(out_row0(k, ch), OROWS)], sem_o)

    def wait_out(k, b):
        for ch in range(C):
            pltpu.make_async_copy(och_v.at[b, ch],
                                  out_h.at[pl.ds(out_row0(k, ch), OROWS)],
                                  sem_o).wait()

    fire_in(0, 0)
    wait_in(0)
    compute(0)
    fire_gather(0)
    fire_in(1, 1)

    @pl.loop(0, NCHUNK // 2)
    def _p1(m):
        for half in range(2):
            k = m * 2 + half
            b = half
            nb = 1 - half
            wait_in(nb)
            compute(nb)
            drain_gather(b)

            @pl.when(k >= 2)
            def _():
                wait_out(k - 2, b)
            fire_gather(nb)
            fire_in((k + 2) % NCHUNK, b)
            deinterleave(b)
            fire_out(k, b)

    wait_in(1)
    wait_out(NCHUNK - 2, 0)
    wait_out(NCHUNK - 1, 1)
    drain_gather(0)


_grid_sample_sc = pl.kernel(
    _body,
    out_type=(
        jax.ShapeDtypeStruct((N * C * VOL // 128, 128), jnp.float32),
        jax.ShapeDtypeStruct((LOCS, 8), jnp.float32),
    ),
    mesh=plsc.VectorSubcoreMesh(core_axis_name="c", subcore_axis_name="s"),
    compiler_params=pltpu.CompilerParams(use_tc_tiling_on_sc=False, needs_layout_passes=False),
    scratch_types=[
        pltpu.VMEM((2, 6, CHUNK), jnp.float32),
        pltpu.VMEM((2, ROWS, 128), jnp.int32),
        pltpu.VMEM((2, CHUNK, 8), jnp.float32),
        pltpu.VMEM((2, C, OROWS, 128), jnp.float32),
        pltpu.SemaphoreType.DMA,
        pltpu.SemaphoreType.DMA,
        pltpu.SemaphoreType.DMA,
    ],
)


def kernel(x, flow, sample_grid):
    assert x.shape == (N, C, D, H, W)
    x2 = x.reshape(N * C, VOL)
    fx = flow[..., 0].reshape(-1)
    fy = flow[..., 1].reshape(-1)
    fz = flow[..., 2].reshape(-1)
    sx = sample_grid[..., 0].reshape(-1)
    sy = sample_grid[..., 1].reshape(-1)
    sz = sample_grid[..., 2].reshape(-1)
    out, _ = _grid_sample_sc(x2, fx, fy, fz, sx, sy, sz)
    return out.reshape(N, C, D, H, W)
